# baseline (speedup 1.0000x reference)
import jax
import jax.numpy as jnp
from jax import lax
from jax.experimental import pallas as pl
from jax.experimental.pallas import tpu as pltpu
from jax.experimental.pallas import tpu_sc as plsc

B = 16
L_KEEP = 2048
N = 4096
D = 512

_LANES = 16
_NPHASE = 8
_TROWS = 256
_UROWS = 16
_NSLOT = 4
_NCHUNK = _TROWS // _UROWS
_TBLROWS = 2057
_IDXW = 264


def _lane0(vec, iota):
    return jnp.sum(jnp.where(iota == 0, vec, 0))


def _lane(vec, j, iota):
    return jnp.sum(jnp.where(iota == j, vec, 0))


def _revert_body(val_hbm, mask_hbm, idx_hbm, rpm_hbm, pos_hbm, out_hbm,
                 idx_v, rpm_v, gidx_v, gbuf, obuf, tbl_s,
                 usem0, usem1, usem2, usem3, psem0, psem1,
                 osem0, osem1, esem):
    usems = (usem0, usem1, usem2, usem3)
    psems = (psem0, psem1)
    osems = (osem0, osem1)
    cid = lax.axis_index("c")
    sid = lax.axis_index("s")
    iota = lax.iota(jnp.int32, _LANES)
    jbase = sid * _TROWS
    off0 = jnp.where(sid == 0, -1, 7)

    def _redirect(idxg):
        inb = idxg < L_KEEP
        idxc = jnp.minimum(jnp.maximum(idxg, 0), L_KEEP - 1)
        rpmg = plsc.load_gather(rpm_v, [idxc])
        keep = inb & (rpmg == 1)
        return jnp.where(keep, 9 + idxg, 0)

    def fire_pos(c, par):
        pltpu.async_copy(
            pos_hbm.at[pl.ds(jbase + c * _UROWS, _UROWS)],
            obuf.at[pl.ds(par * _UROWS, _UROWS)], psems[par])

    def drain_pos(par):
        pltpu.make_async_copy(
            pos_hbm.at[pl.ds(0, _UROWS)],
            obuf.at[pl.ds(par * _UROWS, _UROWS)], psems[par]).wait()

    def drain_out(par):
        pltpu.make_async_copy(
            pos_hbm.at[pl.ds(0, _UROWS)],
            obuf.at[pl.ds(par * _UROWS, _UROWS)], osems[par]).wait()

    def fire_unit(c, slot):
        idxvec = gidx_v[pl.ds(c * _UROWS, _UROWS)]
        for j in range(_UROWS):
            rid = _lane(idxvec, j, iota)
            pltpu.async_copy(
                tbl_s.at[pl.ds(rid, 1)],
                gbuf.at[pl.ds(slot * _UROWS + j, 1)], usems[slot])

    def drain_unit(slot):
        pltpu.make_async_copy(
            pos_hbm.at[pl.ds(0, _UROWS)],
            gbuf.at[pl.ds(slot * _UROWS, _UROWS)], usems[slot]).wait()

    fire_pos(0, 0)
    fire_pos(1, 1)

    def phase_body(p, carry):
        bp = cid * _NPHASE + p

        plsc.subcore_barrier()
        pltpu.sync_copy(val_hbm.at[bp, pl.ds(sid * 128, 128)],
                        tbl_s.at[pl.ds(8 + sid * 128, 128)])

        @pl.when(sid == 0)
        def _stage_rest():
            pltpu.sync_copy(val_hbm.at[bp, pl.ds(L_KEEP, 1)],
                            tbl_s.at[pl.ds(8 + L_KEEP, 1)])
            pltpu.sync_copy(mask_hbm.at[pl.ds(0, 1)],
                            tbl_s.at[pl.ds(0, 1)])

        src0 = bp * N + sid * _TROWS - jnp.where(sid == 0, 0, 8)
        pltpu.sync_copy(idx_hbm.at[pl.ds(src0, _IDXW)], idx_v)
        pltpu.sync_copy(rpm_hbm.at[pl.ds(bp * L_KEEP, L_KEEP)], rpm_v)
        for q in range(_TROWS // _LANES):
            ids = iota + (q * _LANES + off0)
            idxg = plsc.load_gather(idx_v, [jnp.maximum(ids, 0)])
            grow = _redirect(idxg)
            j_abs = iota + (jbase + q * _LANES)
            grow = jnp.where(j_abs == 0, 8, grow)
            gidx_v[pl.ds(q * _LANES, _LANES)] = grow

        plsc.subcore_barrier()

        for s in range(_NSLOT):
            fire_unit(s, s)

        def super_body(m, mcarry):
            for u in range(_NSLOT):
                c = _NSLOT * m + u
                par = u % 2
                slot = u
                drain_unit(slot)
                drain_pos(par)

                obase = par * _UROWS

                @plsc.parallel_loop(0, _UROWS, unroll=4)
                def _add(r, slot=slot, obase=obase):
                    rows_g = jnp.full((_LANES,), slot * _UROWS + r,
                                      jnp.int32)
                    rows_o = jnp.full((_LANES,), obase + r, jnp.int32)
                    for g in range(D // _LANES):
                        cols = iota + g * _LANES
                        x = plsc.load_gather(gbuf, [rows_g, cols])
                        plsc.addupdate_scatter(obuf, [rows_o, cols], x)

                pltpu.async_copy(
                    obuf.at[pl.ds(par * _UROWS, _UROWS)],
                    out_hbm.at[bp, pl.ds(jbase + c * _UROWS, _UROWS)],
                    osems[par])

                @pl.when(c + _NSLOT < _NCHUNK)
                def _refill():
                    fire_unit(c + _NSLOT, slot)

                @pl.when(c + 2 < _NCHUNK)
                def _next_pos():
                    drain_out(par)
                    fire_pos(c + 2, par)
            return mcarry

        lax.fori_loop(0, _NCHUNK // _NSLOT, super_body, 0)
        drain_out(0)
        drain_out(1)

        @pl.when(sid == 15)
        def _last_row():
            idxg = plsc.load_gather(
                idx_v, [jnp.full((_LANES,), _IDXW - 1, jnp.int32)])
            grow = _redirect(idxg)
            rid = _lane0(grow, iota)
            pltpu.async_copy(tbl_s.at[pl.ds(rid, 1)],
                             gbuf.at[pl.ds(0, 1)], esem).wait()
            pltpu.sync_copy(pos_hbm.at[pl.ds(N, 1)], obuf.at[pl.ds(0, 1)])
            rows = jnp.full((_LANES,), 0, jnp.int32)
            for g in range(D // _LANES):
                cols = iota + g * _LANES
                x = plsc.load_gather(gbuf, [rows, cols])
                plsc.addupdate_scatter(obuf, [rows, cols], x)
            pltpu.sync_copy(obuf.at[pl.ds(0, 1)],
                            out_hbm.at[bp, pl.ds(N, 1)])

        @pl.when(p + 1 < _NPHASE)
        def _prefill_next():
            fire_pos(0, 0)
            fire_pos(1, 1)
        return carry

    lax.fori_loop(0, _NPHASE, phase_body, 0)


@jax.jit
def kernel(val, mask_token, remain_padding_mask, revert_idx, pos_emb):
    idx_flat = revert_idx.reshape(B * N).astype(jnp.int32)
    rpm_flat = remain_padding_mask.reshape(B * L_KEEP).astype(jnp.int32)
    pos2d = pos_emb.reshape(N + 1, D)
    mask2d = mask_token.astype(jnp.float32)

    mesh = plsc.VectorSubcoreMesh(core_axis_name="c", subcore_axis_name="s")
    run = pl.kernel(
        _revert_body,
        out_type=jax.ShapeDtypeStruct((B, N + 1, D), jnp.float32),
        mesh=mesh,
        compiler_params=pltpu.CompilerParams(
            needs_layout_passes=False, use_tc_tiling_on_sc=True),
        scratch_types=[
            pltpu.VMEM((_IDXW,), jnp.int32),
            pltpu.VMEM((L_KEEP,), jnp.int32),
            pltpu.VMEM((_TROWS,), jnp.int32),
            pltpu.VMEM((_NSLOT * _UROWS, D), jnp.float32),
            pltpu.VMEM((2 * _UROWS, D), jnp.float32),
            pltpu.VMEM_SHARED((_TBLROWS, D), jnp.float32),
        ] + [pltpu.SemaphoreType.DMA] * 9,
    )
    return run(val, mask2d, idx_flat, rpm_flat, pos2d)


# speedup vs baseline: 1.0273x; 1.0273x over previous
import jax
import jax.numpy as jnp
from jax import lax
from jax.experimental import pallas as pl
from jax.experimental.pallas import tpu as pltpu
from jax.experimental.pallas import tpu_sc as plsc

B = 16
L_KEEP = 2048
N = 4096
D = 512

_LANES = 16
_NPHASE = 8
_TROWS = 256
_UROWS = 16
_NSLOT = 4
_NCHUNK = _TROWS // _UROWS
_TBLROWS = 2057
_IDXW = 264


def _lane0(vec, iota):
    return jnp.sum(jnp.where(iota == 0, vec, 0))


def _lane(vec, j, iota):
    return jnp.sum(jnp.where(iota == j, vec, 0))


def _revert_body(val_hbm, mask_hbm, idx_hbm, rpm_hbm, pos_hbm, out_hbm,
                 idx_v, rpm_v, gidx_v, gbuf, obuf, tbl_s,
                 usem0, usem1, usem2, usem3, psem0, psem1,
                 osem0, osem1, esem):
    usems = (usem0, usem1, usem2, usem3)
    psems = (psem0, psem1)
    osems = (osem0, osem1)
    cid = lax.axis_index("c")
    sid = lax.axis_index("s")
    iota = lax.iota(jnp.int32, _LANES)
    jbase = sid * _TROWS
    off0 = jnp.where(sid == 0, -1, 7)

    def _redirect(idxg):
        inb = idxg < L_KEEP
        idxc = jnp.minimum(jnp.maximum(idxg, 0), L_KEEP - 1)
        rpmg = plsc.load_gather(rpm_v, [idxc])
        keep = inb & (rpmg == 1)
        return jnp.where(keep, 9 + idxg, 0)

    def fire_pos(c, par):
        pltpu.async_copy(
            pos_hbm.at[pl.ds(jbase + c * _UROWS, _UROWS)],
            obuf.at[pl.ds(par * _UROWS, _UROWS)], psems[par])

    def drain_pos(par):
        pltpu.make_async_copy(
            pos_hbm.at[pl.ds(0, _UROWS)],
            obuf.at[pl.ds(par * _UROWS, _UROWS)], psems[par]).wait()

    def drain_out(par):
        pltpu.make_async_copy(
            pos_hbm.at[pl.ds(0, _UROWS)],
            obuf.at[pl.ds(par * _UROWS, _UROWS)], osems[par]).wait()

    def fire_unit(c, slot):
        idxvec = gidx_v[pl.ds(c * _UROWS, _UROWS)]
        for j in range(_UROWS):
            rid = _lane(idxvec, j, iota)
            pltpu.async_copy(
                tbl_s.at[pl.ds(rid, 1)],
                gbuf.at[pl.ds(slot * _UROWS + j, 1)], usems[slot])

    def drain_unit(slot):
        pltpu.make_async_copy(
            pos_hbm.at[pl.ds(0, _UROWS)],
            gbuf.at[pl.ds(slot * _UROWS, _UROWS)], usems[slot]).wait()

    fire_pos(0, 0)
    fire_pos(1, 1)

    def phase_body(p, carry):
        bp = cid * _NPHASE + p

        plsc.subcore_barrier()
        pltpu.sync_copy(val_hbm.at[bp, pl.ds(sid * 128, 128)],
                        tbl_s.at[pl.ds(8 + sid * 128, 128)])

        @pl.when(sid == 0)
        def _stage_rest():
            pltpu.sync_copy(val_hbm.at[bp, pl.ds(L_KEEP, 1)],
                            tbl_s.at[pl.ds(8 + L_KEEP, 1)])
            pltpu.sync_copy(mask_hbm.at[pl.ds(0, 1)],
                            tbl_s.at[pl.ds(0, 1)])

        src0 = bp * N + sid * _TROWS - jnp.where(sid == 0, 0, 8)
        pltpu.sync_copy(idx_hbm.at[pl.ds(src0, _IDXW)], idx_v)
        pltpu.sync_copy(rpm_hbm.at[pl.ds(bp * L_KEEP, L_KEEP)], rpm_v)
        for q in range(_TROWS // _LANES):
            ids = iota + (q * _LANES + off0)
            idxg = plsc.load_gather(idx_v, [jnp.maximum(ids, 0)])
            grow = _redirect(idxg)
            j_abs = iota + (jbase + q * _LANES)
            grow = jnp.where(j_abs == 0, 8, grow)
            gidx_v[pl.ds(q * _LANES, _LANES)] = grow

        plsc.subcore_barrier()

        for s in range(_NSLOT):
            fire_unit(s, s)

        def super_body(m, mcarry):
            for u in range(_NSLOT):
                c = _NSLOT * m + u
                par = u % 2
                slot = u
                drain_unit(slot)
                drain_pos(par)

                obase = par * _UROWS

                @plsc.parallel_loop(0, _UROWS, unroll=2)
                def _add(r, slot=slot, obase=obase):
                    rows_g = jnp.full((_LANES,), slot * _UROWS + r,
                                      jnp.int32)
                    rows_o = jnp.full((_LANES,), obase + r, jnp.int32)
                    for g in range(D // _LANES):
                        cols = iota + g * _LANES
                        x = plsc.load_gather(gbuf, [rows_g, cols])
                        plsc.addupdate_scatter(obuf, [rows_o, cols], x)

                pltpu.async_copy(
                    obuf.at[pl.ds(par * _UROWS, _UROWS)],
                    out_hbm.at[bp, pl.ds(jbase + c * _UROWS, _UROWS)],
                    osems[par])

                @pl.when(c + _NSLOT < _NCHUNK)
                def _refill():
                    fire_unit(c + _NSLOT, slot)

                @pl.when(c + 2 < _NCHUNK)
                def _next_pos():
                    drain_out(par)
                    fire_pos(c + 2, par)
            return mcarry

        lax.fori_loop(0, _NCHUNK // _NSLOT, super_body, 0)
        drain_out(0)
        drain_out(1)

        @pl.when(sid == 15)
        def _last_row():
            idxg = plsc.load_gather(
                idx_v, [jnp.full((_LANES,), _IDXW - 1, jnp.int32)])
            grow = _redirect(idxg)
            rid = _lane0(grow, iota)
            pltpu.async_copy(tbl_s.at[pl.ds(rid, 1)],
                             gbuf.at[pl.ds(0, 1)], esem).wait()
            pltpu.sync_copy(pos_hbm.at[pl.ds(N, 1)], obuf.at[pl.ds(0, 1)])
            rows = jnp.full((_LANES,), 0, jnp.int32)
            for g in range(D // _LANES):
                cols = iota + g * _LANES
                x = plsc.load_gather(gbuf, [rows, cols])
                plsc.addupdate_scatter(obuf, [rows, cols], x)
            pltpu.sync_copy(obuf.at[pl.ds(0, 1)],
                            out_hbm.at[bp, pl.ds(N, 1)])

        @pl.when(p + 1 < _NPHASE)
        def _prefill_next():
            fire_pos(0, 0)
            fire_pos(1, 1)
        return carry

    lax.fori_loop(0, _NPHASE, phase_body, 0)


@jax.jit
def kernel(val, mask_token, remain_padding_mask, revert_idx, pos_emb):
    idx_flat = revert_idx.reshape(B * N).astype(jnp.int32)
    rpm_flat = remain_padding_mask.reshape(B * L_KEEP).astype(jnp.int32)
    pos2d = pos_emb.reshape(N + 1, D)
    mask2d = mask_token.astype(jnp.float32)

    mesh = plsc.VectorSubcoreMesh(core_axis_name="c", subcore_axis_name="s")
    run = pl.kernel(
        _revert_body,
        out_type=jax.ShapeDtypeStruct((B, N + 1, D), jnp.float32),
        mesh=mesh,
        compiler_params=pltpu.CompilerParams(
            needs_layout_passes=False, use_tc_tiling_on_sc=True),
        scratch_types=[
            pltpu.VMEM((_IDXW,), jnp.int32),
            pltpu.VMEM((L_KEEP,), jnp.int32),
            pltpu.VMEM((_TROWS,), jnp.int32),
            pltpu.VMEM((_NSLOT * _UROWS, D), jnp.float32),
            pltpu.VMEM((2 * _UROWS, D), jnp.float32),
            pltpu.VMEM_SHARED((_TBLROWS, D), jnp.float32),
        ] + [pltpu.SemaphoreType.DMA] * 9,
    )
    return run(val, mask2d, idx_flat, rpm_flat, pos2d)
